# Initial kernel scaffold; baseline (speedup 1.0000x reference)
import jax
import jax.numpy as jnp
from jax.experimental import pallas as pl


def kernel(x, wte, W, b):
    raise NotImplementedError("write your pallas kernel here")


# speedup vs baseline: 5.8230x; 5.8230x over previous
import functools

import jax
import jax.numpy as jnp
from jax import lax
from jax.experimental import pallas as pl
from jax.experimental.pallas import tpu as pltpu
from jax.experimental.pallas import tpu_sc as plsc

_D = 16
_GB = 128
_NG = 16
_CB = _GB * _NG


def _table_body(wte_ref, w_ref, b_ref, out_ref):
    out_ref[...] = lax.dot_general(
        wte_ref[...], w_ref[...],
        (((1,), (1,)), ((), ())),
        preferred_element_type=jnp.float32,
    ) + b_ref[...]


def _fused_table(wte, W, b):
    v, d = wte.shape
    return pl.pallas_call(
        _table_body,
        out_shape=jax.ShapeDtypeStruct((v, d), jnp.float32),
    )(wte, W, b.reshape(1, d))


@functools.cache
def _make_gather(B, NC, NS):
    NW = NC * NS
    R = B // NW
    NCH = R // _CB
    assert R % _CB == 0 and B % _GB == 0
    mesh = plsc.VectorSubcoreMesh(core_axis_name="c", subcore_axis_name="s")

    @functools.partial(
        pl.kernel,
        mesh=mesh,
        compiler_params=pltpu.CompilerParams(use_tc_tiling_on_sc=False),
        out_type=jax.ShapeDtypeStruct((B, _D), jnp.float32),
        scratch_types=[
            pltpu.VMEM((_NG, _GB), jnp.int32),
            pltpu.VMEM((_CB, _D), jnp.float32),
            pltpu.SemaphoreType.DMA,
        ],
    )
    def gather_kernel(idx_hbm, table_hbm, out_hbm, idx_v, rows_v, sem):
        wid = lax.axis_index("s") * NC + lax.axis_index("c")
        idx_row0 = wid * (R // _GB)
        out_row0 = wid * R

        def chunk(c, carry):
            pltpu.sync_copy(idx_hbm.at[pl.ds(idx_row0 + c * _NG, _NG)], idx_v)
            copies = [
                pltpu.async_copy(
                    table_hbm.at[idx_v.at[j]],
                    rows_v.at[pl.ds(j * _GB, _GB)],
                    sem,
                )
                for j in range(_NG)
            ]
            for cp in copies:
                cp.wait()
            pltpu.sync_copy(rows_v, out_hbm.at[pl.ds(out_row0 + c * _CB, _CB)])
            return carry

        lax.fori_loop(0, NCH, chunk, 0)

    return gather_kernel


def kernel(x, wte, W, b):
    table = _fused_table(wte, W, b)
    Bt, L = x.shape
    B = Bt * L
    info = plsc.get_sparse_core_info()
    idx2 = x.reshape(B // _GB, _GB).astype(jnp.int32)
    out = _make_gather(B, info.num_cores, info.num_subcores)(idx2, table)
    return out.reshape(Bt, L, _D)


# speedup vs baseline: 6.6286x; 1.1384x over previous
import functools

import jax
import jax.numpy as jnp
from jax import lax
from jax.experimental import pallas as pl
from jax.experimental.pallas import tpu as pltpu
from jax.experimental.pallas import tpu_sc as plsc

_D = 16
_GB = 128
_NG = 16
_CB = _GB * _NG


def _table_body(wte_ref, w_ref, b_ref, out_ref):
    out_ref[...] = lax.dot_general(
        wte_ref[...], w_ref[...],
        (((1,), (1,)), ((), ())),
        preferred_element_type=jnp.float32,
    ) + b_ref[...]


def _fused_table(wte, W, b):
    v, d = wte.shape
    return pl.pallas_call(
        _table_body,
        out_shape=jax.ShapeDtypeStruct((v, d), jnp.float32),
    )(wte, W, b.reshape(1, d))


@functools.cache
def _make_gather(B, V, NC, NS):
    NW = NC * NS
    R = B // NW
    NCH = R // _CB
    assert R % _CB == 0 and B % _GB == 0 and NCH % 2 == 0
    mesh = plsc.VectorSubcoreMesh(core_axis_name="c", subcore_axis_name="s")

    @functools.partial(
        pl.kernel,
        mesh=mesh,
        compiler_params=pltpu.CompilerParams(use_tc_tiling_on_sc=False),
        out_type=jax.ShapeDtypeStruct((B, _D), jnp.float32),
        scratch_types=[
            pltpu.VMEM_SHARED((V, _D), jnp.float32),
            pltpu.VMEM((2, _NG, _GB), jnp.int32),
            pltpu.VMEM((2, _CB, _D), jnp.float32),
            pltpu.SemaphoreType.DMA,
            pltpu.SemaphoreType.DMA,
            pltpu.SemaphoreType.DMA,
        ],
    )
    def gather_kernel(idx_hbm, table_hbm, out_hbm, table_sp, idx_v, rows_v,
                      sem_g, sem_o0, sem_o1):
        cid = lax.axis_index("c")
        sid = lax.axis_index("s")
        wid = sid * NC + cid
        idx_row0 = wid * (R // _GB)
        out_row0 = wid * R
        out_sems = (sem_o0, sem_o1)

        @pl.when(sid == 0)
        def _():
            pltpu.sync_copy(table_hbm, table_sp)

        plsc.subcore_barrier()

        def run_chunk(c, b):
            pltpu.sync_copy(idx_hbm.at[pl.ds(idx_row0 + c * _NG, _NG)],
                            idx_v.at[b])
            copies = [
                pltpu.async_copy(
                    table_sp.at[idx_v.at[b, j]],
                    rows_v.at[b, pl.ds(j * _GB, _GB)],
                    sem_g,
                )
                for j in range(_NG)
            ]
            for cp in copies:
                cp.wait()
            pltpu.async_copy(rows_v.at[b],
                             out_hbm.at[pl.ds(out_row0 + c * _CB, _CB)],
                             out_sems[b])

        def drain_out(c, b):
            pltpu.make_async_copy(
                rows_v.at[b],
                out_hbm.at[pl.ds(out_row0 + c * _CB, _CB)],
                out_sems[b],
            ).wait()

        run_chunk(0, 0)
        run_chunk(1, 1)

        def body(i, carry):
            for b in (0, 1):
                c = 2 * i + b
                drain_out(c - 2, b)
                run_chunk(c, b)
            return carry

        lax.fori_loop(1, NCH // 2, body, 0)
        drain_out(NCH - 2, 0)
        drain_out(NCH - 1, 1)

    return gather_kernel


def kernel(x, wte, W, b):
    table = _fused_table(wte, W, b)
    Bt, L = x.shape
    B = Bt * L
    info = plsc.get_sparse_core_info()
    idx2 = x.reshape(B // _GB, _GB).astype(jnp.int32)
    out = _make_gather(B, wte.shape[0], info.num_cores, info.num_subcores)(
        idx2, table)
    return out.reshape(Bt, L, _D)


# speedup vs baseline: 25.5457x; 3.8538x over previous
import functools

import jax
import jax.numpy as jnp
from jax import lax
from jax.experimental import pallas as pl
from jax.experimental.pallas import tpu as pltpu
from jax.experimental.pallas import tpu_sc as plsc

_D = 16
_LB = 40


def _table_body(wte_ref, w_ref, b_ref, out_ref):
    out_ref[...] = lax.dot_general(
        w_ref[...], wte_ref[...],
        (((1,), (1,)), ((), ())),
        preferred_element_type=jnp.float32,
    ) + b_ref[...]


def _fused_table_t(wte, W, b):
    v, d = wte.shape
    return pl.pallas_call(
        _table_body,
        out_shape=jax.ShapeDtypeStruct((d, v), jnp.float32),
    )(wte, W, b.reshape(d, 1))


@functools.cache
def _make_gather(Bt, L, V, NC, NS):
    NW = NC * NS
    SB = Bt // NW
    NGRP = SB // 16
    assert Bt % NW == 0 and L % _LB == 0 and L % 2 == 0 and SB % 16 == 0
    mesh = plsc.VectorSubcoreMesh(core_axis_name="c", subcore_axis_name="s")

    @functools.partial(
        pl.kernel,
        mesh=mesh,
        compiler_params=pltpu.CompilerParams(needs_layout_passes=False),
        out_type=jax.ShapeDtypeStruct((L * _D, Bt), jnp.float32),
        scratch_types=[
            pltpu.VMEM((_D * V,), jnp.float32),
            pltpu.VMEM((_LB, SB), jnp.int32),
            pltpu.VMEM((2, _D, SB), jnp.float32),
            pltpu.SemaphoreType.DMA,
            pltpu.SemaphoreType.DMA,
        ],
    )
    def gather_kernel(xt_hbm, tab_hbm, out_hbm, tab_v, idx_v, ob_v,
                      sem_o0, sem_o1):
        cid = lax.axis_index("c")
        sid = lax.axis_index("s")
        wid = sid * NC + cid
        b0 = pl.multiple_of(wid * SB, SB)
        out_sems = (sem_o0, sem_o1)

        pltpu.sync_copy(tab_hbm, tab_v)

        def stage_block(l0):
            pltpu.sync_copy(
                xt_hbm.at[pl.ds(l0, _LB), pl.ds(b0, SB)], idx_v)

        def compute_l(l, buf):
            ll = lax.rem(l, _LB)
            obuf = ob_v.at[buf]

            def group(g, carry):
                idx = idx_v[ll, pl.ds(g * 16, 16)]
                for f in range(_D):
                    vals = plsc.load_gather(tab_v, [idx + (f * V)])
                    obuf[f, pl.ds(g * 16, 16)] = vals
                return carry

            lax.fori_loop(0, NGRP, group, 0)
            pltpu.async_copy(
                ob_v.at[buf],
                out_hbm.at[pl.ds(pl.multiple_of(l * _D, _D), _D),
                           pl.ds(b0, SB)],
                out_sems[buf],
            )

        def drain_out(l, buf):
            pltpu.make_async_copy(
                ob_v.at[buf],
                out_hbm.at[pl.ds(pl.multiple_of(l * _D, _D), _D),
                           pl.ds(b0, SB)],
                out_sems[buf],
            ).wait()

        stage_block(0)
        compute_l(0, 0)
        compute_l(1, 1)

        def body(i, carry):
            l0 = 2 * i

            @pl.when(lax.rem(l0, _LB) == 0)
            def _():
                stage_block(pl.multiple_of(l0, _LB))

            for buf in (0, 1):
                l = l0 + buf
                drain_out(l - 2, buf)
                compute_l(l, buf)
            return carry

        lax.fori_loop(1, L // 2, body, 0)
        drain_out(L - 2, 0)
        drain_out(L - 1, 1)

    return gather_kernel


def kernel(x, wte, W, b):
    tableT = _fused_table_t(wte, W, b)
    Bt, L = x.shape
    info = plsc.get_sparse_core_info()
    xt = x.T.astype(jnp.int32)
    out = _make_gather(Bt, L, wte.shape[0], info.num_cores,
                       info.num_subcores)(xt, tableT.reshape(-1))
    return out.reshape(L, _D, Bt).transpose(2, 0, 1)


# speedup vs baseline: 91.6504x; 3.5877x over previous
import functools

import jax
import jax.numpy as jnp
from jax import lax
from jax.experimental import pallas as pl
from jax.experimental.pallas import tpu as pltpu
from jax.experimental.pallas import tpu_sc as plsc

_D = 16
_LB = 40


def _table_body(wte_ref, w_ref, b_ref, out_ref):
    out_ref[...] = lax.dot_general(
        w_ref[...], wte_ref[...],
        (((1,), (1,)), ((), ())),
        preferred_element_type=jnp.float32,
    ) + b_ref[...]


def _fused_table_t(wte, W, b):
    v, d = wte.shape
    return pl.pallas_call(
        _table_body,
        out_shape=jax.ShapeDtypeStruct((d, v), jnp.float32),
    )(wte, W, b.reshape(d, 1))


@functools.cache
def _make_gather(Bt, L, V, NC, NS):
    NW = NC * NS
    SB = Bt // NW
    NGRP = SB // 16
    assert Bt % NW == 0 and L % _LB == 0 and L % 2 == 0 and SB % 16 == 0
    mesh = plsc.VectorSubcoreMesh(core_axis_name="c", subcore_axis_name="s")

    @functools.partial(
        pl.kernel,
        mesh=mesh,
        compiler_params=pltpu.CompilerParams(needs_layout_passes=False),
        out_type=jax.ShapeDtypeStruct((L * _D, Bt), jnp.float32),
        scratch_types=[
            pltpu.VMEM((_D * V,), jnp.float32),
            pltpu.VMEM((_LB, SB), jnp.int32),
            pltpu.VMEM((2, _D, SB), jnp.float32),
            pltpu.SemaphoreType.DMA,
            pltpu.SemaphoreType.DMA,
        ],
    )
    def gather_kernel(xt_hbm, tab_hbm, out_hbm, tab_v, idx_v, ob_v,
                      sem_o0, sem_o1):
        cid = lax.axis_index("c")
        sid = lax.axis_index("s")
        wid = sid * NC + cid
        b0 = pl.multiple_of(wid * SB, SB)
        out_sems = (sem_o0, sem_o1)

        pltpu.sync_copy(tab_hbm, tab_v)

        def stage_block(l0):
            pltpu.sync_copy(
                xt_hbm.at[pl.ds(l0, _LB), pl.ds(b0, SB)], idx_v)

        def compute_l(l, buf):
            ll = lax.rem(l, _LB)
            obuf = ob_v.at[buf]

            @plsc.parallel_loop(0, SB, step=16, unroll=4)
            def group(o):
                idx = idx_v[ll, pl.ds(o, 16)]
                for f in range(_D):
                    vals = plsc.load_gather(tab_v, [idx + (f * V)])
                    obuf[f, pl.ds(o, 16)] = vals
            pltpu.async_copy(
                ob_v.at[buf],
                out_hbm.at[pl.ds(pl.multiple_of(l * _D, _D), _D),
                           pl.ds(b0, SB)],
                out_sems[buf],
            )

        def drain_out(l, buf):
            pltpu.make_async_copy(
                ob_v.at[buf],
                out_hbm.at[pl.ds(pl.multiple_of(l * _D, _D), _D),
                           pl.ds(b0, SB)],
                out_sems[buf],
            ).wait()

        stage_block(0)
        compute_l(0, 0)
        compute_l(1, 1)

        def body(i, carry):
            l0 = 2 * i

            @pl.when(lax.rem(l0, _LB) == 0)
            def _():
                stage_block(pl.multiple_of(l0, _LB))

            for buf in (0, 1):
                l = l0 + buf
                drain_out(l - 2, buf)
                compute_l(l, buf)
            return carry

        lax.fori_loop(1, L // 2, body, 0)
        drain_out(L - 2, 0)
        drain_out(L - 1, 1)

    return gather_kernel


def kernel(x, wte, W, b):
    tableT = _fused_table_t(wte, W, b)
    Bt, L = x.shape
    info = plsc.get_sparse_core_info()
    xt = x.T.astype(jnp.int32)
    out = _make_gather(Bt, L, wte.shape[0], info.num_cores,
                       info.num_subcores)(xt, tableT.reshape(-1))
    return out.reshape(L, _D, Bt).transpose(2, 0, 1)
